# baseline (speedup 1.0000x reference)
import functools

import jax
import jax.numpy as jnp
from jax import lax
from jax.experimental import pallas as pl
from jax.experimental.pallas import tpu as pltpu
from jax.experimental.pallas import tpu_sc as plsc

THRESHOLD = 1e-3
MAX_STEPS = 50

SC_CORES = 2
SC_SUBCORES = 16
NW = SC_CORES * SC_SUBCORES
CHUNK = 128


def _mm_body(x_ref, w_ref, o_ref):
    o_ref[...] = jnp.dot(x_ref[...], w_ref[...],
                         preferred_element_type=jnp.float32)


def _matmul(x, w):
    m, k = x.shape
    p = w.shape[1]
    bm = 1000 if m % 1000 == 0 else m
    return pl.pallas_call(
        _mm_body,
        grid=(m // bm,),
        in_specs=[pl.BlockSpec((bm, k), lambda i: (i, 0)),
                  pl.BlockSpec((k, p), lambda i: (0, 0))],
        out_specs=pl.BlockSpec((bm, p), lambda i: (i, 0)),
        out_shape=jax.ShapeDtypeStruct((m, p), jnp.float32),
    )(x, w)


def _combine_body(u_ref, p_ref, xold_ref, xnew_ref, ss_ref):
    i = pl.program_id(0)
    z = u_ref[...] + p_ref[0] + p_ref[1]
    xn = jnp.tanh(z)
    xnew_ref[...] = xn
    d = xn - xold_ref[...]
    s = jnp.sum(d * d)

    @pl.when(i == 0)
    def _():
        ss_ref[0, 0] = s

    @pl.when(i != 0)
    def _():
        ss_ref[0, 0] += s


def _combine(u, parts, x_old):
    n, h = u.shape
    bm = 1000 if n % 1000 == 0 else n
    return pl.pallas_call(
        _combine_body,
        grid=(n // bm,),
        in_specs=[pl.BlockSpec((bm, h), lambda i: (i, 0)),
                  pl.BlockSpec((SC_CORES, bm, h), lambda i: (0, i, 0)),
                  pl.BlockSpec((bm, h), lambda i: (i, 0))],
        out_specs=[pl.BlockSpec((bm, h), lambda i: (i, 0)),
                   pl.BlockSpec(memory_space=pltpu.SMEM)],
        out_shape=[jax.ShapeDtypeStruct((n, h), jnp.float32),
                   jax.ShapeDtypeStruct((1, 1), jnp.float32)],
    )(u, parts, x_old)


def _readout_body(x_ref, w_ref, b_ref, o_ref):
    o_ref[...] = jnp.dot(x_ref[...], w_ref[...],
                         preferred_element_type=jnp.float32) + b_ref[...]


def _readout(x, w, b):
    m, k = x.shape
    c = w.shape[1]
    bm = 1000 if m % 1000 == 0 else m
    return pl.pallas_call(
        _readout_body,
        grid=(m // bm,),
        in_specs=[pl.BlockSpec((bm, k), lambda i: (i, 0)),
                  pl.BlockSpec((k, c), lambda i: (0, 0)),
                  pl.BlockSpec((1, c), lambda i: (0, 0))],
        out_specs=pl.BlockSpec((bm, c), lambda i: (i, 0)),
        out_shape=jax.ShapeDtypeStruct((m, c), jnp.float32),
    )(x, w, b.reshape(1, c))


GR = 1


def _make_sc_agg(n_nodes, h, ngroup, acc_rows):
    mesh = plsc.VectorSubcoreMesh(core_axis_name="c", subcore_axis_name="s",
                                  num_cores=SC_CORES,
                                  num_subcores=SC_SUBCORES)
    rows_per_sub = acc_rows // SC_SUBCORES
    zrows = 16

    @functools.partial(
        pl.kernel,
        out_type=jax.ShapeDtypeStruct((SC_CORES, acc_rows, h), jnp.float32),
        mesh=mesh,
        scratch_types=[
            pltpu.VMEM((ngroup, CHUNK), jnp.int32),
            pltpu.VMEM((ngroup, CHUNK), jnp.int32),
            pltpu.VMEM((GR * CHUNK, h), jnp.float32),
            pltpu.VMEM((GR * CHUNK, h), jnp.float32),
            pltpu.VMEM((zrows, h), jnp.float32),
            pltpu.VMEM_SHARED((acc_rows, h), jnp.float32),
            pltpu.SemaphoreType.DMA,
        ],
    )
    def sc_agg(xh_hbm, src_hbm, dst_hbm, out_hbm,
               src_v, dst_v, rows0, rows1, zbuf, acc_sh, gsem):
        c = lax.axis_index("c")
        s = lax.axis_index("s")
        wid = c * SC_SUBCORES + s

        zero16 = jnp.zeros((16,), jnp.float32)

        @pl.loop(0, zrows)
        def _(r):
            for k in range(h // 16):
                zbuf[r, pl.ds(k * 16, 16)] = zero16

        @pl.loop(0, rows_per_sub // zrows)
        def _(i):
            pltpu.sync_copy(
                zbuf, acc_sh.at[pl.ds(s * rows_per_sub + i * zrows, zrows)])

        pltpu.sync_copy(src_hbm.at[wid], src_v)
        pltpu.sync_copy(dst_hbm.at[wid], dst_v)
        plsc.subcore_barrier()

        bufs = (rows0, rows1)

        pltpu.async_copy(xh_hbm.at[src_v.at[0]], rows0, gsem)

        @pl.loop(0, ngroup, step=2)
        def _(j):
            for b in range(2):
                cur, nxt = bufs[b], bufs[1 - b]
                pltpu.make_async_copy(
                    xh_hbm.at[src_v.at[j + b]], cur, gsem).wait()

                @pl.when(j + b + 1 < ngroup)
                def _():
                    pltpu.async_copy(
                        xh_hbm.at[src_v.at[j + b + 1]], nxt, gsem)

                pltpu.sync_copy(cur, acc_sh.at[dst_v.at[j + b]], add=True)
                pltpu.async_copy(xh_hbm.at[src_v.at[j + b]], cur, gsem)
                pltpu.make_async_copy(
                    xh_hbm.at[src_v.at[j + b]], cur, gsem).wait()

        plsc.subcore_barrier()
        pltpu.sync_copy(
            acc_sh.at[pl.ds(s * rows_per_sub, rows_per_sub)],
            out_hbm.at[c].at[pl.ds(s * rows_per_sub, rows_per_sub)])

    return sc_agg


def kernel(x, edge_index, W_in, W_h, W_out, b_out):
    n, d = x.shape
    h = W_h.shape[0]
    e = edge_index.shape[1]

    group = GR * CHUNK
    per_w = -(-e // (NW * group)) * group
    ngroup = per_w // group
    if ngroup % 2:
        ngroup += 1
        per_w = ngroup * group
    e_pad = per_w * NW
    acc_rows = -(-(n + 1) // (64 * SC_SUBCORES)) * (64 * SC_SUBCORES)

    src = jnp.concatenate(
        [edge_index[0], jnp.zeros((e_pad - e,), jnp.int32)])
    dst = jnp.concatenate(
        [edge_index[1], jnp.full((e_pad - e,), n, jnp.int32)])
    srcs = src.reshape(NW, ngroup, CHUNK)
    dsts = dst.reshape(NW, ngroup, CHUNK)

    u_proj = _matmul(x, W_in.T)
    sc_agg = _make_sc_agg(n, h, ngroup, acc_rows)

    def cond(state):
        _, norm, steps = state
        return jnp.logical_and(norm > THRESHOLD, steps > 0)

    def body(state):
        x_old, _, steps = state
        xh = _matmul(x_old, W_h.T)
        parts = sc_agg(xh, srcs, dsts)
        x_new, ss = _combine(u_proj, parts, x_old)
        return (x_new, jnp.sqrt(ss[0, 0]), steps - 1)

    x0 = jnp.zeros((n, h), jnp.float32)
    state0 = (x0, jnp.array(jnp.inf, jnp.float32),
              jnp.array(MAX_STEPS, jnp.int32))
    x_final, _, _ = lax.while_loop(cond, body, state0)

    return _readout(x_final, W_out.T, b_out)


# speedup vs baseline: 3.3889x; 3.3889x over previous
import dataclasses
import functools

import jax
import jax.numpy as jnp
from jax import lax
from jax.experimental import pallas as pl
from jax.experimental.pallas import tpu as pltpu
from jax.experimental.pallas import tpu_sc as plsc

THRESHOLD = 1e-3
MAX_STEPS = 50

SC_CORES = 2
SC_SUBCORES = 16
NW = SC_CORES * SC_SUBCORES
CHUNK = 128
SLAB = 32


def _mm_body(x_ref, w_ref, o_ref):
    o_ref[...] = jnp.dot(x_ref[...], w_ref[...],
                         preferred_element_type=jnp.float32)


def _matmul(x, w, out_rows=None):
    m, k = x.shape
    p = w.shape[1]
    bm = 1000 if m % 1000 == 0 else m
    return pl.pallas_call(
        _mm_body,
        grid=(m // bm,),
        in_specs=[pl.BlockSpec((bm, k), lambda i: (i, 0)),
                  pl.BlockSpec((k, p), lambda i: (0, 0))],
        out_specs=pl.BlockSpec((bm, p), lambda i: (i, 0)),
        out_shape=jax.ShapeDtypeStruct((out_rows or m, p), jnp.float32),
    )(x, w)


def _combine_body(u_ref, p_ref, xold_ref, xnew_ref, ss_ref):
    i = pl.program_id(0)
    z = u_ref[...] + p_ref[0] + p_ref[1]
    xn = jnp.tanh(z)
    xnew_ref[...] = xn
    d = xn - xold_ref[...]
    s = jnp.sum(d * d)

    @pl.when(i == 0)
    def _():
        ss_ref[0, 0] = s

    @pl.when(i != 0)
    def _():
        ss_ref[0, 0] += s


def _combine(u, parts, x_old):
    n, h = u.shape
    bm = 1000 if n % 1000 == 0 else n
    return pl.pallas_call(
        _combine_body,
        grid=(n // bm,),
        in_specs=[pl.BlockSpec((bm, h), lambda i: (i, 0)),
                  pl.BlockSpec((SC_CORES, bm, h), lambda i: (0, i, 0)),
                  pl.BlockSpec((bm, h), lambda i: (i, 0))],
        out_specs=[pl.BlockSpec((bm, h), lambda i: (i, 0)),
                   pl.BlockSpec(memory_space=pltpu.SMEM)],
        out_shape=[jax.ShapeDtypeStruct((n, h), jnp.float32),
                   jax.ShapeDtypeStruct((1, 1), jnp.float32)],
    )(u, parts, x_old)


def _readout_body(x_ref, w_ref, b_ref, o_ref):
    o_ref[...] = jnp.dot(x_ref[...], w_ref[...],
                         preferred_element_type=jnp.float32) + b_ref[...]


def _readout(x, w, b):
    m, k = x.shape
    c = w.shape[1]
    bm = 1000 if m % 1000 == 0 else m
    return pl.pallas_call(
        _readout_body,
        grid=(m // bm,),
        in_specs=[pl.BlockSpec((bm, k), lambda i: (i, 0)),
                  pl.BlockSpec((k, c), lambda i: (0, 0)),
                  pl.BlockSpec((1, c), lambda i: (0, 0))],
        out_specs=pl.BlockSpec((bm, c), lambda i: (i, 0)),
        out_shape=jax.ShapeDtypeStruct((m, c), jnp.float32),
    )(x, w, b.reshape(1, c))


def _make_sc_agg(h, ngroup, acc_rows):
    mesh = plsc.VectorSubcoreMesh(core_axis_name="c", subcore_axis_name="s",
                                  num_cores=SC_CORES,
                                  num_subcores=SC_SUBCORES)
    rows_per_sub = acc_rows // SC_SUBCORES
    zrows = 16
    cp = pltpu.CompilerParams()
    if "needs_layout_passes" in pltpu.CompilerParams.__dataclass_fields__:
        cp = dataclasses.replace(cp, needs_layout_passes=False)

    @functools.partial(
        pl.kernel,
        compiler_params=cp,
        out_type=jax.ShapeDtypeStruct((SC_CORES, acc_rows, h), jnp.float32),
        mesh=mesh,
        scratch_types=[
            pltpu.VMEM((ngroup, CHUNK), jnp.int32),
            pltpu.VMEM((ngroup, CHUNK), jnp.int32),
            pltpu.VMEM((ngroup, CHUNK), jnp.int32),
            pltpu.VMEM_SHARED((SC_SUBCORES * SLAB, h), jnp.float32),
            pltpu.VMEM((CHUNK, h), jnp.float32),
            pltpu.VMEM((zrows, h), jnp.float32),
            pltpu.VMEM((ngroup, 16), jnp.int32),
            pltpu.VMEM((ngroup, 16), jnp.int32),
            pltpu.VMEM_SHARED((acc_rows, h), jnp.float32),
            pltpu.SemaphoreType.DMA,
        ],
    )
    def sc_agg(xh_hbm, src_hbm, loc_hbm, dst_hbm, lo_hbm, ok_hbm, out_hbm,
               src_v, loc_v, dst_v, slab, exp, zbuf, lo_sm, ok_sm,
               acc_sh, gsem):
        c = lax.axis_index("c")
        s = lax.axis_index("s")
        wid = c * SC_SUBCORES + s

        zero16 = jnp.zeros((16,), jnp.float32)

        @pl.loop(0, zrows)
        def _(r):
            for k in range(h // 16):
                zbuf[r, pl.ds(k * 16, 16)] = zero16

        @pl.loop(0, rows_per_sub // zrows)
        def _(i):
            pltpu.sync_copy(
                zbuf, acc_sh.at[pl.ds(s * rows_per_sub + i * zrows, zrows)])

        pltpu.sync_copy(src_hbm.at[wid], src_v)
        pltpu.sync_copy(loc_hbm.at[wid], loc_v)
        pltpu.sync_copy(dst_hbm.at[wid], dst_v)
        pltpu.sync_copy(lo_hbm.at[wid], lo_sm)
        pltpu.sync_copy(ok_hbm.at[wid], ok_sm)
        plsc.subcore_barrier()

        @pl.loop(0, ngroup)
        def _(j):
            ok = jnp.max(ok_sm[j, :])

            @pl.when(ok == 1)
            def _():
                lo = pl.multiple_of(jnp.max(lo_sm[j, :]), 8)
                pltpu.sync_copy(xh_hbm.at[pl.ds(lo, SLAB)],
                                slab.at[pl.ds(pl.multiple_of(s * SLAB, 8),
                                              SLAB)])
                pltpu.sync_copy(slab.at[loc_v.at[j]], exp)

            @pl.when(ok == 0)
            def _():
                pltpu.sync_copy(xh_hbm.at[src_v.at[j]], exp)

            pltpu.sync_copy(exp, acc_sh.at[dst_v.at[j]], add=True)

        plsc.subcore_barrier()
        pltpu.sync_copy(
            acc_sh.at[pl.ds(s * rows_per_sub, rows_per_sub)],
            out_hbm.at[c].at[pl.ds(s * rows_per_sub, rows_per_sub)])

    return sc_agg


def kernel(x, edge_index, W_in, W_h, W_out, b_out):
    n, d = x.shape
    h = W_h.shape[0]
    e = edge_index.shape[1]

    group = CHUNK
    per_w = -(-e // (NW * group)) * group
    ngroup = per_w // group
    e_pad = per_w * NW
    acc_rows = -(-(n + 1 + CHUNK) // (16 * SC_SUBCORES)) * (16 * SC_SUBCORES)

    order = jnp.argsort(edge_index[0])
    src_s = edge_index[0][order]
    dst_s = edge_index[1][order]
    pad = e_pad - e
    src_p = jnp.concatenate([src_s, jnp.broadcast_to(src_s[-1:], (pad,))])
    junk = n + 8 + (jnp.arange(pad, dtype=jnp.int32) % CHUNK)
    dst_p = jnp.concatenate([dst_s, junk])

    src_g = src_p.reshape(NW, ngroup, CHUNK)
    dst_g = dst_p.reshape(NW, ngroup, CHUNK)
    lo_g = (src_g[:, :, 0] // 8) * 8
    loc_g = src_g - lo_g[:, :, None]
    ok_g = (loc_g[:, :, -1] < SLAB).astype(jnp.int32)
    loc_g = jnp.minimum(loc_g, SLAB - 1)
    sub_of_w = jnp.arange(NW, dtype=jnp.int32) % SC_SUBCORES
    loc_g = loc_g + (sub_of_w * SLAB)[:, None, None]
    lo_b = jnp.broadcast_to(lo_g[:, :, None], (NW, ngroup, 16))
    ok_b = jnp.broadcast_to(ok_g[:, :, None], (NW, ngroup, 16))

    u_proj = _matmul(x, W_in.T)
    sc_agg = _make_sc_agg(h, ngroup, acc_rows)
    xh_rows = -(-(n + SLAB) // 8) * 8

    def cond(state):
        _, norm, steps = state
        return jnp.logical_and(norm > THRESHOLD, steps > 0)

    def body(state):
        x_old, _, steps = state
        xh = _matmul(x_old, W_h.T, out_rows=xh_rows)
        parts = sc_agg(xh, src_g, loc_g, dst_g, lo_b, ok_b)
        x_new, ss = _combine(u_proj, parts, x_old)
        return (x_new, jnp.sqrt(ss[0, 0]), steps - 1)

    x0 = jnp.zeros((n, h), jnp.float32)
    state0 = (x0, jnp.array(jnp.inf, jnp.float32),
              jnp.array(MAX_STEPS, jnp.int32))
    x_final, _, _ = lax.while_loop(cond, body, state0)

    return _readout(x_final, W_out.T, b_out)


# speedup vs baseline: 3.5456x; 1.0462x over previous
import dataclasses
import functools

import jax
import jax.numpy as jnp
from jax import lax
from jax.experimental import pallas as pl
from jax.experimental.pallas import tpu as pltpu
from jax.experimental.pallas import tpu_sc as plsc

THRESHOLD = 1e-3
MAX_STEPS = 50

SC_CORES = 2
SC_SUBCORES = 16
NW = SC_CORES * SC_SUBCORES
CHUNK = 128
SLAB = 32


def _mm_body(x_ref, w_ref, o_ref):
    o_ref[...] = jnp.dot(x_ref[...], w_ref[...],
                         preferred_element_type=jnp.float32)


def _matmul(x, w, out_rows=None):
    m, k = x.shape
    p = w.shape[1]
    bm = 1000 if m % 1000 == 0 else m
    return pl.pallas_call(
        _mm_body,
        grid=(m // bm,),
        in_specs=[pl.BlockSpec((bm, k), lambda i: (i, 0)),
                  pl.BlockSpec((k, p), lambda i: (0, 0))],
        out_specs=pl.BlockSpec((bm, p), lambda i: (i, 0)),
        out_shape=jax.ShapeDtypeStruct((out_rows or m, p), jnp.float32),
    )(x, w)


def _combine_body(u_ref, p_ref, xold_ref, xnew_ref, ss_ref):
    i = pl.program_id(0)
    z = u_ref[...] + p_ref[0] + p_ref[1]
    xn = jnp.tanh(z)
    xnew_ref[...] = xn
    d = xn - xold_ref[...]
    s = jnp.sum(d * d)

    @pl.when(i == 0)
    def _():
        ss_ref[0, 0] = s

    @pl.when(i != 0)
    def _():
        ss_ref[0, 0] += s


def _combine(u, parts, x_old):
    n, h = u.shape
    bm = 1000 if n % 1000 == 0 else n
    return pl.pallas_call(
        _combine_body,
        grid=(n // bm,),
        in_specs=[pl.BlockSpec((bm, h), lambda i: (i, 0)),
                  pl.BlockSpec((SC_CORES, bm, h), lambda i: (0, i, 0)),
                  pl.BlockSpec((bm, h), lambda i: (i, 0))],
        out_specs=[pl.BlockSpec((bm, h), lambda i: (i, 0)),
                   pl.BlockSpec(memory_space=pltpu.SMEM)],
        out_shape=[jax.ShapeDtypeStruct((n, h), jnp.float32),
                   jax.ShapeDtypeStruct((1, 1), jnp.float32)],
    )(u, parts, x_old)


def _readout_body(x_ref, w_ref, b_ref, o_ref):
    o_ref[...] = jnp.dot(x_ref[...], w_ref[...],
                         preferred_element_type=jnp.float32) + b_ref[...]


def _readout(x, w, b):
    m, k = x.shape
    c = w.shape[1]
    bm = 1000 if m % 1000 == 0 else m
    return pl.pallas_call(
        _readout_body,
        grid=(m // bm,),
        in_specs=[pl.BlockSpec((bm, k), lambda i: (i, 0)),
                  pl.BlockSpec((k, c), lambda i: (0, 0)),
                  pl.BlockSpec((1, c), lambda i: (0, 0))],
        out_specs=pl.BlockSpec((bm, c), lambda i: (i, 0)),
        out_shape=jax.ShapeDtypeStruct((m, c), jnp.float32),
    )(x, w, b.reshape(1, c))


def _make_sc_agg(h, ngroup, acc_rows):
    mesh = plsc.VectorSubcoreMesh(core_axis_name="c", subcore_axis_name="s",
                                  num_cores=SC_CORES,
                                  num_subcores=SC_SUBCORES)
    rows_per_sub = acc_rows // SC_SUBCORES
    zrows = 32
    cp = pltpu.CompilerParams()
    if "needs_layout_passes" in pltpu.CompilerParams.__dataclass_fields__:
        cp = dataclasses.replace(cp, needs_layout_passes=False)

    @functools.partial(
        pl.kernel,
        compiler_params=cp,
        out_type=jax.ShapeDtypeStruct((SC_CORES, acc_rows, h), jnp.float32),
        mesh=mesh,
        scratch_types=[
            pltpu.VMEM((ngroup, CHUNK), jnp.int32),
            pltpu.VMEM((ngroup, CHUNK), jnp.int32),
            pltpu.VMEM_SHARED((SC_SUBCORES * 2 * SLAB, h), jnp.float32),
            pltpu.VMEM((CHUNK, h), jnp.float32),
            pltpu.VMEM((zrows, h), jnp.float32),
            pltpu.VMEM((ngroup, 16), jnp.int32),
            pltpu.VMEM((ngroup, 16), jnp.int32),
            pltpu.VMEM_SHARED((acc_rows, h), jnp.float32),
            pltpu.SemaphoreType.DMA,
            pltpu.SemaphoreType.DMA,
        ],
    )
    def sc_agg(xh_hbm, loc_hbm, dst_hbm, lo_hbm, ok_hbm, out_hbm,
               loc_v, dst_v, slab, exp, zbuf, lo_sm, ok_sm,
               acc_sh, gsem, zsem):
        c = lax.axis_index("c")
        s = lax.axis_index("s")
        wid = c * SC_SUBCORES + s

        zero16 = jnp.zeros((16,), jnp.float32)

        @pl.loop(0, zrows)
        def _(r):
            for k in range(h // 16):
                zbuf[r, pl.ds(k * 16, 16)] = zero16

        @pl.loop(0, rows_per_sub // zrows)
        def _(i):
            pltpu.sync_copy(
                zbuf, acc_sh.at[pl.ds(s * rows_per_sub + i * zrows, zrows)])

        pltpu.sync_copy(loc_hbm.at[wid], loc_v)
        pltpu.sync_copy(dst_hbm.at[wid], dst_v)
        pltpu.sync_copy(lo_hbm.at[wid], lo_sm)
        pltpu.sync_copy(ok_hbm.at[wid], ok_sm)
        plsc.subcore_barrier()

        def slab_region(par):
            return pl.ds(pl.multiple_of((s * 2 + par) * SLAB, 8), SLAB)

        def slab_load(j, par):
            lo = pl.multiple_of(jnp.max(lo_sm[j, :]), 8)
            return pltpu.make_async_copy(xh_hbm.at[pl.ds(lo, SLAB)],
                                         slab.at[slab_region(par)], gsem)

        slab_load(0, 0).start()

        @pl.loop(0, ngroup, step=2)
        def _(j):
            for p in range(2):
                jj = j + p
                slab_load(jj, p).wait()

                @pl.when(jj + 1 < ngroup)
                def _():
                    slab_load(jj + 1, 1 - p).start()

                ok = jnp.max(ok_sm[jj, :])

                @pl.when(ok == 1)
                def _():
                    pltpu.sync_copy(slab.at[loc_v.at[jj]], exp)

                @pl.when(ok == 0)
                def _():
                    pltpu.sync_copy(xh_hbm.at[loc_v.at[jj]], exp)

                pltpu.sync_copy(exp, acc_sh.at[dst_v.at[jj]], add=True)

        plsc.subcore_barrier()
        pltpu.sync_copy(
            acc_sh.at[pl.ds(s * rows_per_sub, rows_per_sub)],
            out_hbm.at[c].at[pl.ds(s * rows_per_sub, rows_per_sub)])

    return sc_agg


def kernel(x, edge_index, W_in, W_h, W_out, b_out):
    n, d = x.shape
    h = W_h.shape[0]
    e = edge_index.shape[1]

    group = CHUNK
    per_w = -(-e // (NW * group)) * group
    ngroup = per_w // group
    e_pad = per_w * NW
    acc_rows = -(-(n + 1 + CHUNK) // (16 * SC_SUBCORES)) * (16 * SC_SUBCORES)

    order = jnp.argsort(edge_index[0])
    src_s = edge_index[0][order]
    dst_s = edge_index[1][order]
    pad = e_pad - e
    src_p = jnp.concatenate([src_s, jnp.broadcast_to(src_s[-1:], (pad,))])
    junk = n + 8 + (jnp.arange(pad, dtype=jnp.int32) % CHUNK)
    dst_p = jnp.concatenate([dst_s, junk])

    src_g = src_p.reshape(NW, ngroup, CHUNK)
    dst_g = dst_p.reshape(NW, ngroup, CHUNK)
    lo_g = (src_g[:, :, 0] // 8) * 8
    loc_g = src_g - lo_g[:, :, None]
    ok_g = (loc_g[:, :, -1] < SLAB).astype(jnp.int32)
    loc_g = jnp.minimum(loc_g, SLAB - 1)
    sub_of_w = jnp.arange(NW, dtype=jnp.int32) % SC_SUBCORES
    par = jnp.arange(ngroup, dtype=jnp.int32) % 2
    loc_g = (loc_g + (sub_of_w[:, None, None] * 2 + par[None, :, None])
             * SLAB)
    loc_g = jnp.where(ok_g[:, :, None] == 1, loc_g, src_g)
    lo_b = jnp.broadcast_to(lo_g[:, :, None], (NW, ngroup, 16))
    ok_b = jnp.broadcast_to(ok_g[:, :, None], (NW, ngroup, 16))

    u_proj = _matmul(x, W_in.T)
    sc_agg = _make_sc_agg(h, ngroup, acc_rows)
    xh_rows = -(-(n + SLAB) // 8) * 8

    def cond(state):
        _, norm, steps = state
        return jnp.logical_and(norm > THRESHOLD, steps > 0)

    def body(state):
        x_old, _, steps = state
        xh = _matmul(x_old, W_h.T, out_rows=xh_rows)
        parts = sc_agg(xh, loc_g, dst_g, lo_b, ok_b)
        x_new, ss = _combine(u_proj, parts, x_old)
        return (x_new, jnp.sqrt(ss[0, 0]), steps - 1)

    x0 = jnp.zeros((n, h), jnp.float32)
    state0 = (x0, jnp.array(jnp.inf, jnp.float32),
              jnp.array(MAX_STEPS, jnp.int32))
    x_final, _, _ = lax.while_loop(cond, body, state0)

    return _readout(x_final, W_out.T, b_out)
